# baseline (speedup 1.0000x reference)
import jax
import jax.numpy as jnp
from jax.experimental import pallas as pl
from jax.experimental.pallas import tpu as pltpu

_N = 64
_NF = 64
_EF = 16
_EFA = _EF + 1
_EFP = 32
_MS = 64
_OF = 64
_PASSES = 3
_Q = 4
_BQ = 4
_NEG = 1024.0


def _mpnn_block(adj_ref, nodes_ref, e4_ref, wmsg_e4_ref, wmsg_n4_ref,
                wupd_h4_ref, wupd_m4_ref, wout_h4_ref, wout_n4_ref, out_ref):
    adj = adj_ref[...].reshape(_BQ, _Q, _N, _N)

    n_r = nodes_ref[...].reshape(_BQ, _Q, _N, _NF)
    nodes_pk = jnp.concatenate([n_r[:, q] for q in range(_Q)],
                               axis=-1)

    e4 = e4_ref[...].reshape(_BQ * _N * _N, _Q * _EFP)
    e_bar = jnp.dot(e4, wmsg_e4_ref[...], preferred_element_type=jnp.float32)
    e_bar = e_bar.reshape(_BQ, _N, _N, _Q * _MS)

    deg = jnp.sum(adj, axis=3)
    deg_bc = jnp.concatenate(
        [jnp.broadcast_to(deg[:, q][..., None], (_BQ, _N, _MS))
         for q in range(_Q)], axis=-1)
    mask = (deg_bc != 0).astype(jnp.float32)
    corr = _N - deg_bc

    hidden = nodes_pk
    for _ in range(_PASSES):
        h_proj = jnp.dot(hidden.reshape(_BQ * _N, _Q * _NF), wmsg_n4_ref[...],
                         preferred_element_type=jnp.float32)
        h_proj = h_proj.reshape(_BQ, 1, _N, _Q * _MS)
        msgs = jnp.sum(jnp.tanh(e_bar + h_proj), axis=2) + corr
        pre = (jnp.dot(hidden.reshape(_BQ * _N, _Q * _NF), wupd_h4_ref[...],
                       preferred_element_type=jnp.float32)
               + jnp.dot(msgs.reshape(_BQ * _N, _Q * _MS), wupd_m4_ref[...],
                         preferred_element_type=jnp.float32))
        upd = jnp.tanh(pre).reshape(_BQ, _N, _Q * _NF)
        hidden = hidden + mask * (upd - hidden)

    h_sum = jnp.sum(hidden * mask, axis=1)
    n_sum = jnp.sum(nodes_pk * mask, axis=1)
    out = (jnp.dot(h_sum, wout_h4_ref[...], preferred_element_type=jnp.float32)
           + jnp.dot(n_sum, wout_n4_ref[...],
                     preferred_element_type=jnp.float32))
    out_ref[...] = out[None]


def _blockdiag4(w):
    r, c = w.shape
    out = jnp.zeros((_Q * r, _Q * c), w.dtype)
    for q in range(_Q):
        out = out.at[q * r:(q + 1) * r, q * c:(q + 1) * c].set(w)
    return out


@jax.jit
def kernel(adjacency, nodes, edges, W_msg, W_upd, W_out):
    b = adjacency.shape[0]
    gb = _Q * _BQ

    e_aug = jnp.concatenate(
        [edges.astype(jnp.bfloat16),
         (1.0 - adjacency)[..., None].astype(jnp.bfloat16)], axis=-1)
    e_aug = jnp.pad(e_aug, ((0, 0), (0, 0), (0, 0), (0, _EFP - _EFA)))
    e4 = e_aug.reshape(b // _Q, _Q, _N, _N, _EFP)
    e4 = jnp.transpose(e4, (0, 2, 3, 1, 4)).reshape(b // _Q, _N,
                                                    _N * _Q * _EFP)

    w_e = jnp.concatenate(
        [W_msg[_NF:], jnp.full((1, _MS), -_NEG, jnp.float32),
         jnp.zeros((_EFP - _EFA, _MS), jnp.float32)], axis=0)
    wmsg_e4 = _blockdiag4(w_e).astype(jnp.bfloat16)
    wmsg_n4 = _blockdiag4(W_msg[:_NF])
    wupd_h4 = _blockdiag4(W_upd[:_NF])
    wupd_m4 = _blockdiag4(W_upd[_NF:])
    wout_h4 = _blockdiag4(W_out[:_NF])
    wout_n4 = _blockdiag4(W_out[_NF:])

    grid = (b // gb,)
    full = lambda i: (0, 0)
    out = pl.pallas_call(
        _mpnn_block,
        grid=grid,
        in_specs=[
            pl.BlockSpec((gb, _N, _N), lambda i: (i, 0, 0)),
            pl.BlockSpec((gb, _N, _NF), lambda i: (i, 0, 0)),
            pl.BlockSpec((_BQ, _N, _N * _Q * _EFP), lambda i: (i, 0, 0)),
            pl.BlockSpec((_Q * _EFP, _Q * _MS), full),
            pl.BlockSpec((_Q * _NF, _Q * _MS), full),
            pl.BlockSpec((_Q * _NF, _Q * _NF), full),
            pl.BlockSpec((_Q * _MS, _Q * _NF), full),
            pl.BlockSpec((_Q * _NF, _Q * _OF), full),
            pl.BlockSpec((_Q * _NF, _Q * _OF), full),
        ],
        out_specs=pl.BlockSpec((1, _BQ, _Q * _OF), lambda i: (i, 0, 0)),
        out_shape=jax.ShapeDtypeStruct((b // gb, _BQ, _Q * _OF), jnp.float32),
        compiler_params=pltpu.CompilerParams(
            dimension_semantics=("arbitrary",),
        ),
    )(adjacency, nodes, e4, wmsg_e4, wmsg_n4, wupd_h4, wupd_m4,
      wout_h4, wout_n4)
    return out.reshape(b, _OF)


# speedup vs baseline: 1.2214x; 1.2214x over previous
import jax
import jax.numpy as jnp
from jax.experimental import pallas as pl
from jax.experimental.pallas import tpu as pltpu

_N = 64
_NF = 64
_EF = 16
_EFA = _EF + 1
_MS = 64
_OF = 64
_PASSES = 3
_GP = 8
_NEG = 1024.0


def _mpnn_block(adj_ref, nodes_ref, edges_ref, wmsg_e0_ref, wmsg_e1_ref,
                wmsg_n2_ref, wupd_h2_ref, wupd_m2_ref, wout_h2_ref,
                wout_n2_ref, out_ref):
    adj = adj_ref[...].reshape(2, _GP, _N, _N)

    n_r = nodes_ref[...].reshape(2, _GP, _N, _NF)
    nodes_pk = jnp.concatenate([n_r[0], n_r[1]], axis=-1)

    e_r = edges_ref[...].reshape(2, _GP, _N, _N, _EFA)
    e0 = e_r[0].reshape(_GP * _N * _N, _EFA)
    e1 = e_r[1].reshape(_GP * _N * _N, _EFA)
    e_bar = (jnp.dot(e0, wmsg_e0_ref[...], preferred_element_type=jnp.float32)
             + jnp.dot(e1, wmsg_e1_ref[...],
                       preferred_element_type=jnp.float32))
    e_bar = e_bar.reshape(_GP, _N, _N, 2 * _MS)

    deg = jnp.sum(adj, axis=3)
    d0 = jnp.broadcast_to(deg[0][..., None], (_GP, _N, _MS))
    d1 = jnp.broadcast_to(deg[1][..., None], (_GP, _N, _MS))
    deg_bc = jnp.concatenate([d0, d1], axis=-1)
    mask = (deg_bc != 0).astype(jnp.float32)
    corr = _N - deg_bc

    hidden = nodes_pk
    for _ in range(_PASSES):
        h_proj = jnp.dot(hidden.reshape(_GP * _N, 2 * _NF), wmsg_n2_ref[...],
                         preferred_element_type=jnp.float32)
        h_proj = h_proj.reshape(_GP, 1, _N, 2 * _MS)
        msgs = jnp.sum(jnp.tanh(e_bar + h_proj), axis=2) + corr
        pre = (jnp.dot(hidden.reshape(_GP * _N, 2 * _NF), wupd_h2_ref[...],
                       preferred_element_type=jnp.float32)
               + jnp.dot(msgs.reshape(_GP * _N, 2 * _MS), wupd_m2_ref[...],
                         preferred_element_type=jnp.float32))
        upd = jnp.tanh(pre).reshape(_GP, _N, 2 * _NF)
        hidden = hidden + mask * (upd - hidden)

    h_sum = jnp.sum(hidden * mask, axis=1)
    n_sum = jnp.sum(nodes_pk * mask, axis=1)
    out = (jnp.dot(h_sum, wout_h2_ref[...], preferred_element_type=jnp.float32)
           + jnp.dot(n_sum, wout_n2_ref[...],
                     preferred_element_type=jnp.float32))
    out_ref[...] = out[None]


def _blockdiag2(w):
    r, c = w.shape
    z = jnp.zeros((r, c), w.dtype)
    return jnp.concatenate(
        [jnp.concatenate([w, z], axis=1), jnp.concatenate([z, w], axis=1)],
        axis=0)


@jax.jit
def kernel(adjacency, nodes, edges, W_msg, W_upd, W_out):
    b = adjacency.shape[0]
    gb = 2 * _GP

    e_aug = jnp.concatenate(
        [edges.astype(jnp.bfloat16),
         (1.0 - adjacency)[..., None].astype(jnp.bfloat16)], axis=-1)
    e_aug = e_aug.reshape(b, _N, _N * _EFA)

    w_e = jnp.concatenate(
        [W_msg[_NF:], jnp.full((1, _MS), -_NEG, jnp.float32)], axis=0)
    z = jnp.zeros((_EFA, _MS), jnp.float32)
    wmsg_e0 = jnp.concatenate([w_e, z], axis=1).astype(jnp.bfloat16)
    wmsg_e1 = jnp.concatenate([z, w_e], axis=1).astype(jnp.bfloat16)
    wmsg_n2 = _blockdiag2(W_msg[:_NF])
    wupd_h2 = _blockdiag2(W_upd[:_NF])
    wupd_m2 = _blockdiag2(W_upd[_NF:])
    wout_h2 = _blockdiag2(W_out[:_NF])
    wout_n2 = _blockdiag2(W_out[_NF:])

    grid = (b // gb,)
    full = lambda i: (0, 0)
    out = pl.pallas_call(
        _mpnn_block,
        grid=grid,
        in_specs=[
            pl.BlockSpec((gb, _N, _N), lambda i: (i, 0, 0)),
            pl.BlockSpec((gb, _N, _NF), lambda i: (i, 0, 0)),
            pl.BlockSpec((gb, _N, _N * _EFA), lambda i: (i, 0, 0)),
            pl.BlockSpec((_EFA, 2 * _MS), full),
            pl.BlockSpec((_EFA, 2 * _MS), full),
            pl.BlockSpec((2 * _NF, 2 * _MS), full),
            pl.BlockSpec((2 * _NF, 2 * _NF), full),
            pl.BlockSpec((2 * _MS, 2 * _NF), full),
            pl.BlockSpec((2 * _NF, 2 * _OF), full),
            pl.BlockSpec((2 * _NF, 2 * _OF), full),
        ],
        out_specs=pl.BlockSpec((1, _GP, 2 * _OF), lambda i: (i, 0, 0)),
        out_shape=jax.ShapeDtypeStruct((b // gb, _GP, 2 * _OF), jnp.float32),
        compiler_params=pltpu.CompilerParams(
            dimension_semantics=("arbitrary",),
        ),
    )(adjacency, nodes, e_aug, wmsg_e0, wmsg_e1, wmsg_n2, wupd_h2, wupd_m2,
      wout_h2, wout_n2)
    out = out.reshape(b // gb, _GP, 2, _OF)
    return jnp.transpose(out, (0, 2, 1, 3)).reshape(b, _OF)
